# speedup vs baseline: 1.0566x; 1.0566x over previous
import functools

import jax
import jax.numpy as jnp
from jax import lax
from jax.experimental import pallas as pl
from jax.experimental.pallas import tpu as pltpu

N = 10000
E = 320000
D = 128
EPS = 1e-5
BLK = 1000
GRID = N // BLK


def _pre_body(mix_ref, gs_ref, struct_ref, svd_ref, wst_ref, bs_ref,
              wgt_ref, bg_ref, lsg_ref, lsb_ref, lvg_ref, lvb_ref,
              sin_ref, tin_ref, out_ref):
    alpha = 1.0 / (1.0 + jnp.exp(-mix_ref[0, 0]))
    ctx = jnp.dot(gs_ref[...], wgt_ref[...],
                  preferred_element_type=jnp.float32) + bg_ref[...]
    sp = jnp.dot(struct_ref[...], wst_ref[...],
                 preferred_element_type=jnp.float32) + bs_ref[...]
    m = jnp.mean(sp, axis=-1, keepdims=True)
    v = jnp.mean((sp - m) ** 2, axis=-1, keepdims=True)
    sn = (sp - m) / jnp.sqrt(v + EPS) * lsg_ref[...] + lsb_ref[...]
    sv = svd_ref[...]
    m2 = jnp.mean(sv, axis=-1, keepdims=True)
    v2 = jnp.mean((sv - m2) ** 2, axis=-1, keepdims=True)
    vn = (sv - m2) / jnp.sqrt(v2 + EPS) * lvg_ref[...] + lvb_ref[...]
    x = alpha * sn + (1.0 - alpha) * vn + ctx
    x = x * sin_ref[...] + tin_ref[...]
    out_ref[...] = jnp.maximum(x, 0.0)


def _layer_body(p0_ref, p1_ref, inv_ref, x_ref, wlt_ref, wrt_ref, c_ref,
                out_ref, *, relu):
    agg = (p0_ref[...] + p1_ref[...]) * inv_ref[...]
    y = (jnp.dot(agg, wlt_ref[...], preferred_element_type=jnp.float32)
         + jnp.dot(x_ref[...], wrt_ref[...],
                   preferred_element_type=jnp.float32)
         + c_ref[...])
    if relu:
        y = jnp.maximum(y, 0.0)
    out_ref[...] = y


def _vec_spec():
    return pl.BlockSpec((1, D), lambda i: (0, 0))


def _pre_call(mix, gs, struct, svd, wst, bs, wgt, bg, lsg, lsb, lvg, lvb,
              sin, tin):
    return pl.pallas_call(
        _pre_body,
        grid=(GRID,),
        in_specs=[
            pl.BlockSpec((1, 1), lambda i: (0, 0)),
            pl.BlockSpec((1, 3), lambda i: (0, 0)),
            pl.BlockSpec((BLK, 6), lambda i: (i, 0)),
            pl.BlockSpec((BLK, D), lambda i: (i, 0)),
            pl.BlockSpec((6, D), lambda i: (0, 0)),
            _vec_spec(), pl.BlockSpec((3, D), lambda i: (0, 0)),
            _vec_spec(), _vec_spec(), _vec_spec(), _vec_spec(), _vec_spec(),
            _vec_spec(), _vec_spec(),
        ],
        out_specs=pl.BlockSpec((BLK, D), lambda i: (i, 0)),
        out_shape=jax.ShapeDtypeStruct((N, D), jnp.float32),
    )(mix, gs, struct, svd, wst, bs, wgt, bg, lsg, lsb, lvg, lvb, sin, tin)


def _layer_call(p0, p1, inv, x, wlt, wrt, c, relu):
    return pl.pallas_call(
        functools.partial(_layer_body, relu=relu),
        grid=(GRID,),
        in_specs=[
            pl.BlockSpec((BLK, D), lambda i: (i, 0)),
            pl.BlockSpec((BLK, D), lambda i: (i, 0)),
            pl.BlockSpec((BLK, 1), lambda i: (i, 0)),
            pl.BlockSpec((BLK, D), lambda i: (i, 0)),
            pl.BlockSpec((D, D), lambda i: (0, 0)),
            pl.BlockSpec((D, D), lambda i: (0, 0)),
            _vec_spec(),
        ],
        out_specs=pl.BlockSpec((BLK, D), lambda i: (i, 0)),
        out_shape=jax.ShapeDtypeStruct((N, D), jnp.float32),
    )(p0, p1, inv, x, wlt, wrt, c)


def _bn_fold(g, b, rm, rv):
    s = g / jnp.sqrt(rv + EPS)
    return s, b - rm * s


def kernel(struct, svd, graph_summary, Ws, bs, Wg, bg, ln_s_g, ln_s_b,
           ln_v_g, ln_v_b, mix, bn_in_g, bn_in_b, bn_in_rm, bn_in_rv,
           Wl0, bl0, Wr0, bn0_g, bn0_b, bn0_rm, bn0_rv,
           Wl1, bl1, Wr1, bn1_g, bn1_b, bn1_rm, bn1_rv,
           Wl2, bl2, Wr2, bn2_g, bn2_b, bn2_rm, bn2_rv, edge_index):
    f32 = jnp.float32
    sin, tin = _bn_fold(bn_in_g, bn_in_b, bn_in_rm, bn_in_rv)
    x = _pre_call(mix.reshape(1, 1), graph_summary.reshape(1, 3),
                  struct, svd, Ws.T, bs.reshape(1, D), Wg.T,
                  bg.reshape(1, D), ln_s_g.reshape(1, D),
                  ln_s_b.reshape(1, D), ln_v_g.reshape(1, D),
                  ln_v_b.reshape(1, D), sin.reshape(1, D),
                  tin.reshape(1, D))

    src = edge_index[0]
    dst = edge_index[1]
    cnt = jax.ops.segment_sum(jnp.ones((E,), f32), dst, num_segments=N)
    inv = (1.0 / jnp.maximum(cnt, 1.0)).reshape(N, 1)
    zeros = jnp.zeros((N, D), f32)

    layers = [(Wl0, bl0, Wr0, bn0_g, bn0_b, bn0_rm, bn0_rv),
              (Wl1, bl1, Wr1, bn1_g, bn1_b, bn1_rm, bn1_rv),
              (Wl2, bl2, Wr2, bn2_g, bn2_b, bn2_rm, bn2_rv)]
    for i, (Wl, bl, Wr, g, b, rm, rv) in enumerate(layers):
        s, t = _bn_fold(g, b, rm, rv)
        wlt = Wl.T * s[None, :]
        wrt = Wr.T * s[None, :]
        c = (bl * s + t).reshape(1, D)
        ssum = jax.ops.segment_sum(x[src], dst, num_segments=N)
        x = _layer_call(ssum, zeros, inv, x, wlt, wrt, c, relu=(i < 2))
    return x


# speedup vs baseline: 4.2332x; 4.0065x over previous
import functools

import jax
import jax.numpy as jnp
from jax import lax
from jax.experimental import pallas as pl
from jax.experimental.pallas import tpu as pltpu
from jax.experimental.pallas import tpu_sc as plsc

N = 10000
E = 320000
D = 128
EPS = 1e-5
BLK = 1000
GRID = N // BLK

_NC = 2
_NS = 16
_NW = _NC * _NS
_EW = E // _NW
_CH = 80
_NCHUNK = _EW // _CH
_RT = 624
_RTAIL = N - _RT * _NS

_SC_MESH = plsc.VectorSubcoreMesh(core_axis_name="c", subcore_axis_name="s",
                                  num_cores=_NC, num_subcores=_NS)


def _agg_body_common(x_hbm, src_hbm, dst_hbm, zeros_hbm, zeros16_hbm,
                     ones16_hbm, out_hbm, outcnt_hbm, srcv, dstv, rows,
                     ones16, acc, acc_cnt, *, with_cnt):
    c = lax.axis_index("c")
    s = lax.axis_index("s")
    wid = s * _NC + c
    row0 = pl.multiple_of(s * _RT, 8)
    last = s == _NS - 1
    pltpu.sync_copy(zeros_hbm.at[pl.ds(row0, _RT)], acc.at[pl.ds(row0, _RT)])
    if with_cnt:
        pltpu.sync_copy(zeros16_hbm.at[pl.ds(row0, _RT)],
                        acc_cnt.at[pl.ds(row0, _RT)])
        pltpu.sync_copy(ones16_hbm, ones16)

    @pl.when(last)
    def _zero_tail():
        pltpu.sync_copy(zeros_hbm.at[pl.ds(_RT * _NS, _RTAIL)],
                        acc.at[pl.ds(_RT * _NS, _RTAIL)])
        if with_cnt:
            pltpu.sync_copy(zeros16_hbm.at[pl.ds(_RT * _NS, _RTAIL)],
                            acc_cnt.at[pl.ds(_RT * _NS, _RTAIL)])

    plsc.subcore_barrier()

    def chunk(i, carry):
        base = pl.multiple_of(wid * _EW + i * _CH, _CH)
        pltpu.sync_copy(src_hbm.at[pl.ds(base, _CH)], srcv)
        pltpu.sync_copy(dst_hbm.at[pl.ds(base, _CH)], dstv)
        pltpu.sync_copy(x_hbm.at[srcv], rows)
        pltpu.sync_copy(rows, acc.at[dstv], add=True)
        if with_cnt:
            pltpu.sync_copy(ones16, acc_cnt.at[dstv], add=True)
        return carry

    lax.fori_loop(0, _NCHUNK, chunk, 0)
    plsc.subcore_barrier()
    pltpu.sync_copy(acc.at[pl.ds(row0, _RT)],
                    out_hbm.at[c, pl.ds(row0, _RT)])
    if with_cnt:
        pltpu.sync_copy(acc_cnt.at[pl.ds(row0, _RT)],
                        outcnt_hbm.at[c, pl.ds(row0, _RT)])

    @pl.when(last)
    def _write_tail():
        pltpu.sync_copy(acc.at[pl.ds(_RT * _NS, _RTAIL)],
                        out_hbm.at[c, pl.ds(_RT * _NS, _RTAIL)])
        if with_cnt:
            pltpu.sync_copy(acc_cnt.at[pl.ds(_RT * _NS, _RTAIL)],
                            outcnt_hbm.at[c, pl.ds(_RT * _NS, _RTAIL)])


def _agg_body_cnt(x_hbm, src_hbm, dst_hbm, zeros_hbm, zeros16_hbm,
                  ones16_hbm, out_hbm, outcnt_hbm, srcv, dstv, rows,
                  ones16, acc, acc_cnt):
    _agg_body_common(x_hbm, src_hbm, dst_hbm, zeros_hbm, zeros16_hbm,
                     ones16_hbm, out_hbm, outcnt_hbm, srcv, dstv, rows,
                     ones16, acc, acc_cnt, with_cnt=True)


def _agg_body(x_hbm, src_hbm, dst_hbm, zeros_hbm, out_hbm, srcv, dstv,
              rows, acc):
    _agg_body_common(x_hbm, src_hbm, dst_hbm, zeros_hbm, None, None,
                     out_hbm, None, srcv, dstv, rows, None, acc,
                     None, with_cnt=False)


def _cnt_body(ones_hbm, dst_hbm, zeros_hbm, out_hbm, dstv, rows, acc):
    c = lax.axis_index("c")
    s = lax.axis_index("s")
    wid = s * _NC + c
    row0 = pl.multiple_of(s * _RT, 8)
    pltpu.sync_copy(zeros_hbm.at[pl.ds(row0, _RT)], acc.at[pl.ds(row0, _RT)])

    @pl.when(s == _NS - 1)
    def _zero_tail():
        pltpu.sync_copy(zeros_hbm.at[pl.ds(_RT * _NS, _RTAIL)],
                        acc.at[pl.ds(_RT * _NS, _RTAIL)])

    pltpu.sync_copy(ones_hbm, rows)
    plsc.subcore_barrier()

    def chunk(i, carry):
        base = pl.multiple_of(wid * _EW + i * _CH, _CH)
        pltpu.sync_copy(dst_hbm.at[pl.ds(base, _CH)], dstv)
        pltpu.sync_copy(rows, acc.at[dstv], add=True)
        return carry

    lax.fori_loop(0, _NCHUNK, chunk, 0)
    plsc.subcore_barrier()
    pltpu.sync_copy(acc.at[pl.ds(row0, _RT)],
                    out_hbm.at[c, pl.ds(row0, _RT)])

    @pl.when(s == _NS - 1)
    def _write_tail():
        pltpu.sync_copy(acc.at[pl.ds(_RT * _NS, _RTAIL)],
                        out_hbm.at[c, pl.ds(_RT * _NS, _RTAIL)])


_cnt_call = pl.kernel(
    _cnt_body,
    out_type=jax.ShapeDtypeStruct((_NC, N, D), jnp.float32),
    mesh=_SC_MESH,
    scratch_types=[
        pltpu.VMEM((_CH,), jnp.int32),
        pltpu.VMEM((_CH, D), jnp.float32),
        pltpu.VMEM_SHARED((N, D), jnp.float32),
    ],
)


def _gather_body(x_hbm, src_hbm, out_hbm, srcv, rows, sem):
    c = lax.axis_index("c")
    s = lax.axis_index("s")
    wid = s * _NC + c

    def chunk(i, carry):
        base = pl.multiple_of(wid * _EW + i * _CH, _CH)
        pltpu.sync_copy(src_hbm.at[pl.ds(base, _CH)], srcv)
        pltpu.async_copy(x_hbm.at[srcv], rows, sem).wait()
        pltpu.sync_copy(rows, out_hbm.at[pl.ds(base, _CH)])
        return carry

    lax.fori_loop(0, _NCHUNK, chunk, 0)


_gather_call = pl.kernel(
    _gather_body,
    out_type=jax.ShapeDtypeStruct((E, D), jnp.float32),
    mesh=_SC_MESH,
    scratch_types=[
        pltpu.VMEM((_CH,), jnp.int32),
        pltpu.VMEM((_CH, D), jnp.float32),
        pltpu.SemaphoreType.DMA,
    ],
)


_agg_cnt_call = pl.kernel(
    _agg_body_cnt,
    out_type=(jax.ShapeDtypeStruct((_NC, N, D), jnp.float32),
              jax.ShapeDtypeStruct((_NC, N, 16), jnp.float32)),
    mesh=_SC_MESH,
    scratch_types=[
        pltpu.VMEM((_CH,), jnp.int32),
        pltpu.VMEM((_CH,), jnp.int32),
        pltpu.VMEM((_CH, D), jnp.float32),
        pltpu.VMEM((_CH, 16), jnp.float32),
        pltpu.VMEM_SHARED((N, D), jnp.float32),
        pltpu.VMEM_SHARED((N, 16), jnp.float32),
    ],
)

_agg_call = pl.kernel(
    _agg_body,
    out_type=jax.ShapeDtypeStruct((_NC, N, D), jnp.float32),
    mesh=_SC_MESH,
    scratch_types=[
        pltpu.VMEM((_CH,), jnp.int32),
        pltpu.VMEM((_CH,), jnp.int32),
        pltpu.VMEM((_CH, D), jnp.float32),
        pltpu.VMEM_SHARED((N, D), jnp.float32),
    ],
)


def _pre_body(mix_ref, gs_ref, struct_ref, svd_ref, wst_ref, bs_ref,
              wgt_ref, bg_ref, lsg_ref, lsb_ref, lvg_ref, lvb_ref,
              sin_ref, tin_ref, out_ref):
    alpha = 1.0 / (1.0 + jnp.exp(-mix_ref[0, 0]))
    ctx = jnp.dot(gs_ref[...], wgt_ref[...],
                  preferred_element_type=jnp.float32) + bg_ref[...]
    sp = jnp.dot(struct_ref[...], wst_ref[...],
                 preferred_element_type=jnp.float32) + bs_ref[...]
    m = jnp.mean(sp, axis=-1, keepdims=True)
    v = jnp.mean((sp - m) ** 2, axis=-1, keepdims=True)
    sn = (sp - m) / jnp.sqrt(v + EPS) * lsg_ref[...] + lsb_ref[...]
    sv = svd_ref[...]
    m2 = jnp.mean(sv, axis=-1, keepdims=True)
    v2 = jnp.mean((sv - m2) ** 2, axis=-1, keepdims=True)
    vn = (sv - m2) / jnp.sqrt(v2 + EPS) * lvg_ref[...] + lvb_ref[...]
    x = alpha * sn + (1.0 - alpha) * vn + ctx
    x = x * sin_ref[...] + tin_ref[...]
    out_ref[...] = jnp.maximum(x, 0.0)


def _layer_body(p0_ref, p1_ref, inv_ref, x_ref, wlt_ref, wrt_ref, c_ref,
                out_ref, *, relu):
    agg = (p0_ref[...] + p1_ref[...]) * inv_ref[...]
    y = (jnp.dot(agg, wlt_ref[...], preferred_element_type=jnp.float32)
         + jnp.dot(x_ref[...], wrt_ref[...],
                   preferred_element_type=jnp.float32)
         + c_ref[...])
    if relu:
        y = jnp.maximum(y, 0.0)
    out_ref[...] = y


def _vec_spec():
    return pl.BlockSpec((1, D), lambda i: (0, 0))


def _pre_call(mix, gs, struct, svd, wst, bs, wgt, bg, lsg, lsb, lvg, lvb,
              sin, tin):
    return pl.pallas_call(
        _pre_body,
        grid=(GRID,),
        in_specs=[
            pl.BlockSpec((1, 1), lambda i: (0, 0)),
            pl.BlockSpec((1, 3), lambda i: (0, 0)),
            pl.BlockSpec((BLK, 6), lambda i: (i, 0)),
            pl.BlockSpec((BLK, D), lambda i: (i, 0)),
            pl.BlockSpec((6, D), lambda i: (0, 0)),
            _vec_spec(), pl.BlockSpec((3, D), lambda i: (0, 0)),
            _vec_spec(), _vec_spec(), _vec_spec(), _vec_spec(), _vec_spec(),
            _vec_spec(), _vec_spec(),
        ],
        out_specs=pl.BlockSpec((BLK, D), lambda i: (i, 0)),
        out_shape=jax.ShapeDtypeStruct((N, D), jnp.float32),
    )(mix, gs, struct, svd, wst, bs, wgt, bg, lsg, lsb, lvg, lvb, sin, tin)


def _layer_call(p0, p1, inv, x, wlt, wrt, c, relu):
    return pl.pallas_call(
        functools.partial(_layer_body, relu=relu),
        grid=(GRID,),
        in_specs=[
            pl.BlockSpec((BLK, D), lambda i: (i, 0)),
            pl.BlockSpec((BLK, D), lambda i: (i, 0)),
            pl.BlockSpec((BLK, 1), lambda i: (i, 0)),
            pl.BlockSpec((BLK, D), lambda i: (i, 0)),
            pl.BlockSpec((D, D), lambda i: (0, 0)),
            pl.BlockSpec((D, D), lambda i: (0, 0)),
            _vec_spec(),
        ],
        out_specs=pl.BlockSpec((BLK, D), lambda i: (i, 0)),
        out_shape=jax.ShapeDtypeStruct((N, D), jnp.float32),
    )(p0, p1, inv, x, wlt, wrt, c)


def _bn_fold(g, b, rm, rv):
    s = g / jnp.sqrt(rv + EPS)
    return s, b - rm * s


def kernel(struct, svd, graph_summary, Ws, bs, Wg, bg, ln_s_g, ln_s_b,
           ln_v_g, ln_v_b, mix, bn_in_g, bn_in_b, bn_in_rm, bn_in_rv,
           Wl0, bl0, Wr0, bn0_g, bn0_b, bn0_rm, bn0_rv,
           Wl1, bl1, Wr1, bn1_g, bn1_b, bn1_rm, bn1_rv,
           Wl2, bl2, Wr2, bn2_g, bn2_b, bn2_rm, bn2_rv, edge_index):
    f32 = jnp.float32
    sin, tin = _bn_fold(bn_in_g, bn_in_b, bn_in_rm, bn_in_rv)
    x = _pre_call(mix.reshape(1, 1), graph_summary.reshape(1, 3),
                  struct, svd, Ws.T, bs.reshape(1, D), Wg.T,
                  bg.reshape(1, D), ln_s_g.reshape(1, D),
                  ln_s_b.reshape(1, D), ln_v_g.reshape(1, D),
                  ln_v_b.reshape(1, D), sin.reshape(1, D),
                  tin.reshape(1, D))

    src = edge_index[0]
    dst = edge_index[1]
    zeros = jnp.zeros((N, D), f32)
    ones16 = jnp.ones((_CH, 16), f32)

    layers = [(Wl0, bl0, Wr0, bn0_g, bn0_b, bn0_rm, bn0_rv),
              (Wl1, bl1, Wr1, bn1_g, bn1_b, bn1_rm, bn1_rv),
              (Wl2, bl2, Wr2, bn2_g, bn2_b, bn2_rm, bn2_rv)]
    inv = None
    for i, (Wl, bl, Wr, g, b, rm, rv) in enumerate(layers):
        s, t = _bn_fold(g, b, rm, rv)
        wlt = Wl.T * s[None, :]
        wrt = Wr.T * s[None, :]
        c = (bl * s + t).reshape(1, D)
        if i == 0:
            cparts = _cnt_call(jnp.ones((_CH, D), f32), dst, zeros)
            cnt = cparts[0, :, 0] + cparts[1, :, 0]
            inv = (1.0 / jnp.maximum(cnt, 1.0)).reshape(N, 1)
        parts = _agg_call(x, src, dst, zeros)
        x = _layer_call(parts[0], parts[1], inv, x, wlt, wrt, c,
                        relu=(i < 2))
    return x


# speedup vs baseline: 8.2597x; 1.9512x over previous
import functools

import jax
import jax.numpy as jnp
from jax import lax
from jax.experimental import pallas as pl
from jax.experimental.pallas import tpu as pltpu
from jax.experimental.pallas import tpu_sc as plsc

N = 10000
E = 320000
D = 128
EPS = 1e-5
BLK = 1000
GRID = N // BLK

_NC = 2
_NS = 16
_NW = _NC * _NS
_EW = E // _NW
_CH = 80
_NCHUNK = _EW // _CH
_RT = 624
_RTAIL = N - _RT * _NS

_SC_MESH = plsc.VectorSubcoreMesh(core_axis_name="c", subcore_axis_name="s",
                                  num_cores=_NC, num_subcores=_NS)


def _zero_acc(zeros_hbm, acc, row0, s):
    pltpu.sync_copy(zeros_hbm.at[pl.ds(row0, _RT)], acc.at[pl.ds(row0, _RT)])

    @pl.when(s == _NS - 1)
    def _zero_tail():
        pltpu.sync_copy(zeros_hbm.at[pl.ds(_RT * _NS, _RTAIL)],
                        acc.at[pl.ds(_RT * _NS, _RTAIL)])


def _write_acc(acc, out_hbm, row0, c, s):
    pltpu.sync_copy(acc.at[pl.ds(row0, _RT)],
                    out_hbm.at[c, pl.ds(row0, _RT)])

    @pl.when(s == _NS - 1)
    def _write_tail():
        pltpu.sync_copy(acc.at[pl.ds(_RT * _NS, _RTAIL)],
                        out_hbm.at[c, pl.ds(_RT * _NS, _RTAIL)])


def _agg_body(x_hbm, src3_hbm, dst3_hbm, zeros_hbm, out_hbm, src_v, dst_v,
              rows0, rows1, acc, sem0, sem1):
    c = lax.axis_index("c")
    s = lax.axis_index("s")
    wid = s * _NC + c
    row0 = pl.multiple_of(s * _RT, 8)
    _zero_acc(zeros_hbm, acc, row0, s)
    pltpu.sync_copy(src3_hbm.at[wid], src_v)
    plsc.subcore_barrier()

    pltpu.async_copy(x_hbm.at[src_v.at[0]], rows0, sem0)

    def chunk(i, carry):
        even = i % 2 == 0
        nxt = i + 1

        @pl.when(jnp.logical_and(nxt < _NCHUNK, even))
        def _start_odd():
            pltpu.async_copy(x_hbm.at[src_v.at[nxt]], rows1, sem1)

        @pl.when(jnp.logical_and(nxt < _NCHUNK, jnp.logical_not(even)))
        def _start_even():
            pltpu.async_copy(x_hbm.at[src_v.at[nxt]], rows0, sem0)

        pltpu.sync_copy(dst3_hbm.at[wid, i], dst_v)

        @pl.when(even)
        def _drain_even():
            pltpu.make_async_copy(x_hbm.at[src_v.at[i]], rows0, sem0).wait()
            pltpu.sync_copy(rows0, acc.at[dst_v], add=True)

        @pl.when(jnp.logical_not(even))
        def _drain_odd():
            pltpu.make_async_copy(x_hbm.at[src_v.at[i]], rows1, sem1).wait()
            pltpu.sync_copy(rows1, acc.at[dst_v], add=True)

        return carry

    lax.fori_loop(0, _NCHUNK, chunk, 0)
    plsc.subcore_barrier()
    _write_acc(acc, out_hbm, row0, c, s)


def _cnt_body(ones_hbm, dst3_hbm, zeros_hbm, out_hbm, dst_v, rows, acc):
    c = lax.axis_index("c")
    s = lax.axis_index("s")
    wid = s * _NC + c
    row0 = pl.multiple_of(s * _RT, 8)
    _zero_acc(zeros_hbm, acc, row0, s)
    pltpu.sync_copy(ones_hbm, rows)
    pltpu.sync_copy(dst3_hbm.at[wid], dst_v)
    plsc.subcore_barrier()

    def chunk(i, carry):
        pltpu.sync_copy(rows, acc.at[dst_v.at[i]], add=True)
        return carry

    lax.fori_loop(0, _NCHUNK, chunk, 0)
    plsc.subcore_barrier()
    _write_acc(acc, out_hbm, row0, c, s)


_cnt_call = pl.kernel(
    _cnt_body,
    out_type=jax.ShapeDtypeStruct((_NC, N, D), jnp.float32),
    mesh=_SC_MESH,
    scratch_types=[
        pltpu.VMEM((_NCHUNK, _CH), jnp.int32),
        pltpu.VMEM((_CH, D), jnp.float32),
        pltpu.VMEM_SHARED((N, D), jnp.float32),
    ],
)


_agg_call = pl.kernel(
    _agg_body,
    out_type=jax.ShapeDtypeStruct((_NC, N, D), jnp.float32),
    mesh=_SC_MESH,
    scratch_types=[
        pltpu.VMEM((_NCHUNK, _CH), jnp.int32),
        pltpu.VMEM((_CH,), jnp.int32),
        pltpu.VMEM((_CH, D), jnp.float32),
        pltpu.VMEM((_CH, D), jnp.float32),
        pltpu.VMEM_SHARED((N, D), jnp.float32),
        pltpu.SemaphoreType.DMA,
        pltpu.SemaphoreType.DMA,
    ],
)


def _pre_body(mix_ref, gs_ref, struct_ref, svd_ref, wst_ref, bs_ref,
              wgt_ref, bg_ref, lsg_ref, lsb_ref, lvg_ref, lvb_ref,
              sin_ref, tin_ref, out_ref):
    alpha = 1.0 / (1.0 + jnp.exp(-mix_ref[0, 0]))
    ctx = jnp.dot(gs_ref[...], wgt_ref[...],
                  preferred_element_type=jnp.float32) + bg_ref[...]
    sp = jnp.dot(struct_ref[...], wst_ref[...],
                 preferred_element_type=jnp.float32) + bs_ref[...]
    m = jnp.mean(sp, axis=-1, keepdims=True)
    v = jnp.mean((sp - m) ** 2, axis=-1, keepdims=True)
    sn = (sp - m) / jnp.sqrt(v + EPS) * lsg_ref[...] + lsb_ref[...]
    sv = svd_ref[...]
    m2 = jnp.mean(sv, axis=-1, keepdims=True)
    v2 = jnp.mean((sv - m2) ** 2, axis=-1, keepdims=True)
    vn = (sv - m2) / jnp.sqrt(v2 + EPS) * lvg_ref[...] + lvb_ref[...]
    x = alpha * sn + (1.0 - alpha) * vn + ctx
    x = x * sin_ref[...] + tin_ref[...]
    out_ref[...] = jnp.maximum(x, 0.0)


def _layer_body(p0_ref, p1_ref, inv_ref, x_ref, wlt_ref, wrt_ref, c_ref,
                out_ref, *, relu):
    agg = (p0_ref[...] + p1_ref[...]) * inv_ref[...]
    y = (jnp.dot(agg, wlt_ref[...], preferred_element_type=jnp.float32)
         + jnp.dot(x_ref[...], wrt_ref[...],
                   preferred_element_type=jnp.float32)
         + c_ref[...])
    if relu:
        y = jnp.maximum(y, 0.0)
    out_ref[...] = y


def _vec_spec():
    return pl.BlockSpec((1, D), lambda i: (0, 0))


def _pre_call(mix, gs, struct, svd, wst, bs, wgt, bg, lsg, lsb, lvg, lvb,
              sin, tin):
    return pl.pallas_call(
        _pre_body,
        grid=(GRID,),
        in_specs=[
            pl.BlockSpec((1, 1), lambda i: (0, 0)),
            pl.BlockSpec((1, 3), lambda i: (0, 0)),
            pl.BlockSpec((BLK, 6), lambda i: (i, 0)),
            pl.BlockSpec((BLK, D), lambda i: (i, 0)),
            pl.BlockSpec((6, D), lambda i: (0, 0)),
            _vec_spec(), pl.BlockSpec((3, D), lambda i: (0, 0)),
            _vec_spec(), _vec_spec(), _vec_spec(), _vec_spec(), _vec_spec(),
            _vec_spec(), _vec_spec(),
        ],
        out_specs=pl.BlockSpec((BLK, D), lambda i: (i, 0)),
        out_shape=jax.ShapeDtypeStruct((N, D), jnp.float32),
    )(mix, gs, struct, svd, wst, bs, wgt, bg, lsg, lsb, lvg, lvb, sin, tin)


def _layer_call(p0, p1, inv, x, wlt, wrt, c, relu):
    return pl.pallas_call(
        functools.partial(_layer_body, relu=relu),
        grid=(GRID,),
        in_specs=[
            pl.BlockSpec((BLK, D), lambda i: (i, 0)),
            pl.BlockSpec((BLK, D), lambda i: (i, 0)),
            pl.BlockSpec((BLK, 1), lambda i: (i, 0)),
            pl.BlockSpec((BLK, D), lambda i: (i, 0)),
            pl.BlockSpec((D, D), lambda i: (0, 0)),
            pl.BlockSpec((D, D), lambda i: (0, 0)),
            _vec_spec(),
        ],
        out_specs=pl.BlockSpec((BLK, D), lambda i: (i, 0)),
        out_shape=jax.ShapeDtypeStruct((N, D), jnp.float32),
    )(p0, p1, inv, x, wlt, wrt, c)


def _bn_fold(g, b, rm, rv):
    s = g / jnp.sqrt(rv + EPS)
    return s, b - rm * s


def kernel(struct, svd, graph_summary, Ws, bs, Wg, bg, ln_s_g, ln_s_b,
           ln_v_g, ln_v_b, mix, bn_in_g, bn_in_b, bn_in_rm, bn_in_rv,
           Wl0, bl0, Wr0, bn0_g, bn0_b, bn0_rm, bn0_rv,
           Wl1, bl1, Wr1, bn1_g, bn1_b, bn1_rm, bn1_rv,
           Wl2, bl2, Wr2, bn2_g, bn2_b, bn2_rm, bn2_rv, edge_index):
    f32 = jnp.float32
    sin, tin = _bn_fold(bn_in_g, bn_in_b, bn_in_rm, bn_in_rv)
    x = _pre_call(mix.reshape(1, 1), graph_summary.reshape(1, 3),
                  struct, svd, Ws.T, bs.reshape(1, D), Wg.T,
                  bg.reshape(1, D), ln_s_g.reshape(1, D),
                  ln_s_b.reshape(1, D), ln_v_g.reshape(1, D),
                  ln_v_b.reshape(1, D), sin.reshape(1, D),
                  tin.reshape(1, D))

    src3 = edge_index[0].reshape(_NW, _NCHUNK, _CH)
    dst3 = edge_index[1].reshape(_NW, _NCHUNK, _CH)
    zeros = jnp.zeros((N, D), f32)

    layers = [(Wl0, bl0, Wr0, bn0_g, bn0_b, bn0_rm, bn0_rv),
              (Wl1, bl1, Wr1, bn1_g, bn1_b, bn1_rm, bn1_rv),
              (Wl2, bl2, Wr2, bn2_g, bn2_b, bn2_rm, bn2_rv)]
    inv = None
    for i, (Wl, bl, Wr, g, b, rm, rv) in enumerate(layers):
        s, t = _bn_fold(g, b, rm, rv)
        wlt = Wl.T * s[None, :]
        wrt = Wr.T * s[None, :]
        c = (bl * s + t).reshape(1, D)
        if i == 0:
            cparts = _cnt_call(jnp.ones((_CH, D), f32), dst3, zeros)
            cnt = cparts[0, :, 0] + cparts[1, :, 0]
            inv = (1.0 / jnp.maximum(cnt, 1.0)).reshape(N, 1)
        parts = _agg_call(x, src3, dst3, zeros)
        x = _layer_call(parts[0], parts[1], inv, x, wlt, wrt, c,
                        relu=(i < 2))
    return x


# speedup vs baseline: 8.9975x; 1.0893x over previous
import functools

import jax
import jax.numpy as jnp
from jax import lax
from jax.experimental import pallas as pl
from jax.experimental.pallas import tpu as pltpu
from jax.experimental.pallas import tpu_sc as plsc

N = 10000
E = 320000
D = 128
EPS = 1e-5
BLK = 1000
GRID = N // BLK

_NC = 2
_NS = 16
_NW = _NC * _NS
_EW = E // _NW
_CH = 80
_NCHUNK = _EW // _CH
_RT = 624
_RTAIL = N - _RT * _NS

_SC_MESH = plsc.VectorSubcoreMesh(core_axis_name="c", subcore_axis_name="s",
                                  num_cores=_NC, num_subcores=_NS)


def _zero_acc(zeros_hbm, acc, row0, s):
    pltpu.sync_copy(zeros_hbm.at[pl.ds(row0, _RT)], acc.at[pl.ds(row0, _RT)])

    @pl.when(s == _NS - 1)
    def _zero_tail():
        pltpu.sync_copy(zeros_hbm.at[pl.ds(_RT * _NS, _RTAIL)],
                        acc.at[pl.ds(_RT * _NS, _RTAIL)])


def _write_acc(acc, out_hbm, row0, c, s):
    pltpu.sync_copy(acc.at[pl.ds(row0, _RT)],
                    out_hbm.at[c, pl.ds(row0, _RT)])

    @pl.when(s == _NS - 1)
    def _write_tail():
        pltpu.sync_copy(acc.at[pl.ds(_RT * _NS, _RTAIL)],
                        out_hbm.at[c, pl.ds(_RT * _NS, _RTAIL)])


def _agg_body(x_hbm, src3_hbm, dst3_hbm, zeros_hbm, out_hbm, src_v,
              dst0, dst1, rows0, rows1, acc, gsem0, gsem1, dsem0, dsem1):
    c = lax.axis_index("c")
    s = lax.axis_index("s")
    wid = s * _NC + c
    row0 = pl.multiple_of(s * _RT, 8)
    _zero_acc(zeros_hbm, acc, row0, s)
    pltpu.sync_copy(src3_hbm.at[wid], src_v)
    plsc.subcore_barrier()

    pltpu.async_copy(x_hbm.at[src_v.at[0]], rows0, gsem0)
    pltpu.async_copy(dst3_hbm.at[wid, 0], dst0, dsem0)
    pltpu.async_copy(dst3_hbm.at[wid, 1], dst1, dsem1)

    def chunk(i, carry):
        even = i % 2 == 0
        nxt = i + 1

        @pl.when(jnp.logical_and(nxt < _NCHUNK, even))
        def _start_odd():
            pltpu.async_copy(x_hbm.at[src_v.at[nxt]], rows1, gsem1)

        @pl.when(jnp.logical_and(nxt < _NCHUNK, jnp.logical_not(even)))
        def _start_even():
            pltpu.async_copy(x_hbm.at[src_v.at[nxt]], rows0, gsem0)

        @pl.when(even)
        def _drain_even():
            pltpu.make_async_copy(dst3_hbm.at[wid, i], dst0, dsem0).wait()
            pltpu.make_async_copy(x_hbm.at[src_v.at[i]], rows0, gsem0).wait()
            pltpu.sync_copy(rows0, acc.at[dst0], add=True)

            @pl.when(i + 2 < _NCHUNK)
            def _next_dst_even():
                pltpu.async_copy(dst3_hbm.at[wid, i + 2], dst0, dsem0)

        @pl.when(jnp.logical_not(even))
        def _drain_odd():
            pltpu.make_async_copy(dst3_hbm.at[wid, i], dst1, dsem1).wait()
            pltpu.make_async_copy(x_hbm.at[src_v.at[i]], rows1, gsem1).wait()
            pltpu.sync_copy(rows1, acc.at[dst1], add=True)

            @pl.when(i + 2 < _NCHUNK)
            def _next_dst_odd():
                pltpu.async_copy(dst3_hbm.at[wid, i + 2], dst1, dsem1)

        return carry

    lax.fori_loop(0, _NCHUNK, chunk, 0)
    plsc.subcore_barrier()
    _write_acc(acc, out_hbm, row0, c, s)


def _cnt_body(ones_hbm, dst3_hbm, zeros_hbm, out_hbm, dst_v, rows, acc):
    c = lax.axis_index("c")
    s = lax.axis_index("s")
    wid = s * _NC + c
    row0 = pl.multiple_of(s * _RT, 8)
    _zero_acc(zeros_hbm, acc, row0, s)
    pltpu.sync_copy(ones_hbm, rows)
    pltpu.sync_copy(dst3_hbm.at[wid], dst_v)
    plsc.subcore_barrier()

    def chunk(i, carry):
        pltpu.sync_copy(rows, acc.at[dst_v.at[i]], add=True)
        return carry

    lax.fori_loop(0, _NCHUNK, chunk, 0)
    plsc.subcore_barrier()
    _write_acc(acc, out_hbm, row0, c, s)


_cnt_call = pl.kernel(
    _cnt_body,
    out_type=jax.ShapeDtypeStruct((_NC, N, D), jnp.float32),
    mesh=_SC_MESH,
    scratch_types=[
        pltpu.VMEM((_NCHUNK, _CH), jnp.int32),
        pltpu.VMEM((_CH, D), jnp.float32),
        pltpu.VMEM_SHARED((N, D), jnp.float32),
    ],
)


_agg_call = pl.kernel(
    _agg_body,
    out_type=jax.ShapeDtypeStruct((_NC, N, D), jnp.float32),
    mesh=_SC_MESH,
    scratch_types=[
        pltpu.VMEM((_NCHUNK, _CH), jnp.int32),
        pltpu.VMEM((_CH,), jnp.int32),
        pltpu.VMEM((_CH,), jnp.int32),
        pltpu.VMEM((_CH, D), jnp.float32),
        pltpu.VMEM((_CH, D), jnp.float32),
        pltpu.VMEM_SHARED((N, D), jnp.float32),
        pltpu.SemaphoreType.DMA,
        pltpu.SemaphoreType.DMA,
        pltpu.SemaphoreType.DMA,
        pltpu.SemaphoreType.DMA,
    ],
)


def _pre_body(mix_ref, gs_ref, struct_ref, svd_ref, wst_ref, bs_ref,
              wgt_ref, bg_ref, lsg_ref, lsb_ref, lvg_ref, lvb_ref,
              sin_ref, tin_ref, out_ref):
    alpha = 1.0 / (1.0 + jnp.exp(-mix_ref[0, 0]))
    ctx = jnp.dot(gs_ref[...], wgt_ref[...],
                  preferred_element_type=jnp.float32) + bg_ref[...]
    sp = jnp.dot(struct_ref[...], wst_ref[...],
                 preferred_element_type=jnp.float32) + bs_ref[...]
    m = jnp.mean(sp, axis=-1, keepdims=True)
    v = jnp.mean((sp - m) ** 2, axis=-1, keepdims=True)
    sn = (sp - m) / jnp.sqrt(v + EPS) * lsg_ref[...] + lsb_ref[...]
    sv = svd_ref[...]
    m2 = jnp.mean(sv, axis=-1, keepdims=True)
    v2 = jnp.mean((sv - m2) ** 2, axis=-1, keepdims=True)
    vn = (sv - m2) / jnp.sqrt(v2 + EPS) * lvg_ref[...] + lvb_ref[...]
    x = alpha * sn + (1.0 - alpha) * vn + ctx
    x = x * sin_ref[...] + tin_ref[...]
    out_ref[...] = jnp.maximum(x, 0.0)


def _layer_body(p0_ref, p1_ref, inv_ref, x_ref, wlt_ref, wrt_ref, c_ref,
                out_ref, *, relu):
    agg = (p0_ref[...] + p1_ref[...]) * inv_ref[...]
    y = (jnp.dot(agg, wlt_ref[...], preferred_element_type=jnp.float32)
         + jnp.dot(x_ref[...], wrt_ref[...],
                   preferred_element_type=jnp.float32)
         + c_ref[...])
    if relu:
        y = jnp.maximum(y, 0.0)
    out_ref[...] = y


def _vec_spec():
    return pl.BlockSpec((1, D), lambda i: (0, 0))


def _pre_call(mix, gs, struct, svd, wst, bs, wgt, bg, lsg, lsb, lvg, lvb,
              sin, tin):
    return pl.pallas_call(
        _pre_body,
        grid=(GRID,),
        in_specs=[
            pl.BlockSpec((1, 1), lambda i: (0, 0)),
            pl.BlockSpec((1, 3), lambda i: (0, 0)),
            pl.BlockSpec((BLK, 6), lambda i: (i, 0)),
            pl.BlockSpec((BLK, D), lambda i: (i, 0)),
            pl.BlockSpec((6, D), lambda i: (0, 0)),
            _vec_spec(), pl.BlockSpec((3, D), lambda i: (0, 0)),
            _vec_spec(), _vec_spec(), _vec_spec(), _vec_spec(), _vec_spec(),
            _vec_spec(), _vec_spec(),
        ],
        out_specs=pl.BlockSpec((BLK, D), lambda i: (i, 0)),
        out_shape=jax.ShapeDtypeStruct((N, D), jnp.float32),
    )(mix, gs, struct, svd, wst, bs, wgt, bg, lsg, lsb, lvg, lvb, sin, tin)


def _layer_call(p0, p1, inv, x, wlt, wrt, c, relu):
    return pl.pallas_call(
        functools.partial(_layer_body, relu=relu),
        grid=(GRID,),
        in_specs=[
            pl.BlockSpec((BLK, D), lambda i: (i, 0)),
            pl.BlockSpec((BLK, D), lambda i: (i, 0)),
            pl.BlockSpec((BLK, 1), lambda i: (i, 0)),
            pl.BlockSpec((BLK, D), lambda i: (i, 0)),
            pl.BlockSpec((D, D), lambda i: (0, 0)),
            pl.BlockSpec((D, D), lambda i: (0, 0)),
            _vec_spec(),
        ],
        out_specs=pl.BlockSpec((BLK, D), lambda i: (i, 0)),
        out_shape=jax.ShapeDtypeStruct((N, D), jnp.float32),
    )(p0, p1, inv, x, wlt, wrt, c)


def _bn_fold(g, b, rm, rv):
    s = g / jnp.sqrt(rv + EPS)
    return s, b - rm * s


def kernel(struct, svd, graph_summary, Ws, bs, Wg, bg, ln_s_g, ln_s_b,
           ln_v_g, ln_v_b, mix, bn_in_g, bn_in_b, bn_in_rm, bn_in_rv,
           Wl0, bl0, Wr0, bn0_g, bn0_b, bn0_rm, bn0_rv,
           Wl1, bl1, Wr1, bn1_g, bn1_b, bn1_rm, bn1_rv,
           Wl2, bl2, Wr2, bn2_g, bn2_b, bn2_rm, bn2_rv, edge_index):
    f32 = jnp.float32
    sin, tin = _bn_fold(bn_in_g, bn_in_b, bn_in_rm, bn_in_rv)
    x = _pre_call(mix.reshape(1, 1), graph_summary.reshape(1, 3),
                  struct, svd, Ws.T, bs.reshape(1, D), Wg.T,
                  bg.reshape(1, D), ln_s_g.reshape(1, D),
                  ln_s_b.reshape(1, D), ln_v_g.reshape(1, D),
                  ln_v_b.reshape(1, D), sin.reshape(1, D),
                  tin.reshape(1, D))

    src3 = edge_index[0].reshape(_NW, _NCHUNK, _CH)
    dst3 = edge_index[1].reshape(_NW, _NCHUNK, _CH)
    zeros = jnp.zeros((N, D), f32)

    layers = [(Wl0, bl0, Wr0, bn0_g, bn0_b, bn0_rm, bn0_rv),
              (Wl1, bl1, Wr1, bn1_g, bn1_b, bn1_rm, bn1_rv),
              (Wl2, bl2, Wr2, bn2_g, bn2_b, bn2_rm, bn2_rv)]
    inv = None
    for i, (Wl, bl, Wr, g, b, rm, rv) in enumerate(layers):
        s, t = _bn_fold(g, b, rm, rv)
        wlt = Wl.T * s[None, :]
        wrt = Wr.T * s[None, :]
        c = (bl * s + t).reshape(1, D)
        if i == 0:
            cparts = _cnt_call(jnp.ones((_CH, D), f32), dst3, zeros)
            cnt = cparts[0, :, 0] + cparts[1, :, 0]
            inv = (1.0 / jnp.maximum(cnt, 1.0)).reshape(N, 1)
        parts = _agg_call(x, src3, dst3, zeros)
        x = _layer_call(parts[0], parts[1], inv, x, wlt, wrt, c,
                        relu=(i < 2))
    return x


# speedup vs baseline: 9.0077x; 1.0011x over previous
import functools

import jax
import jax.numpy as jnp
from jax import lax
from jax.experimental import pallas as pl
from jax.experimental.pallas import tpu as pltpu
from jax.experimental.pallas import tpu_sc as plsc

N = 10000
E = 320000
D = 128
EPS = 1e-5
BLK = 1000
GRID = N // BLK

_NC = 2
_NS = 16
_NW = _NC * _NS
_EW = E // _NW
_CH = 80
_NCHUNK = _EW // _CH
_RT = 624
_RTAIL = N - _RT * _NS

_SC_MESH = plsc.VectorSubcoreMesh(core_axis_name="c", subcore_axis_name="s",
                                  num_cores=_NC, num_subcores=_NS)


def _zero_acc(zeros_hbm, acc, row0, s):
    pltpu.sync_copy(zeros_hbm.at[pl.ds(row0, _RT)], acc.at[pl.ds(row0, _RT)])

    @pl.when(s == _NS - 1)
    def _zero_tail():
        pltpu.sync_copy(zeros_hbm.at[pl.ds(_RT * _NS, _RTAIL)],
                        acc.at[pl.ds(_RT * _NS, _RTAIL)])


def _write_acc(acc, out_hbm, row0, c, s):
    pltpu.sync_copy(acc.at[pl.ds(row0, _RT)],
                    out_hbm.at[c, pl.ds(row0, _RT)])

    @pl.when(s == _NS - 1)
    def _write_tail():
        pltpu.sync_copy(acc.at[pl.ds(_RT * _NS, _RTAIL)],
                        out_hbm.at[c, pl.ds(_RT * _NS, _RTAIL)])


def _agg_body(x_hbm, src3_hbm, dst3_hbm, zeros_hbm, out_hbm, src_v,
              dst0, dst1, rows0, rows1, acc, gsem0, gsem1, dsem0, dsem1):
    c = lax.axis_index("c")
    s = lax.axis_index("s")
    wid = s * _NC + c
    row0 = pl.multiple_of(s * _RT, 8)
    _zero_acc(zeros_hbm, acc, row0, s)
    pltpu.sync_copy(src3_hbm.at[wid], src_v)
    plsc.subcore_barrier()

    pltpu.async_copy(x_hbm.at[src_v.at[0]], rows0, gsem0)
    pltpu.async_copy(dst3_hbm.at[wid, 0], dst0, dsem0)
    pltpu.async_copy(dst3_hbm.at[wid, 1], dst1, dsem1)

    def chunk(i, carry):
        even = i % 2 == 0
        nxt = i + 1

        @pl.when(jnp.logical_and(nxt < _NCHUNK, even))
        def _start_odd():
            pltpu.async_copy(x_hbm.at[src_v.at[nxt]], rows1, gsem1)

        @pl.when(jnp.logical_and(nxt < _NCHUNK, jnp.logical_not(even)))
        def _start_even():
            pltpu.async_copy(x_hbm.at[src_v.at[nxt]], rows0, gsem0)

        @pl.when(even)
        def _drain_even():
            pltpu.make_async_copy(dst3_hbm.at[wid, i], dst0, dsem0).wait()
            pltpu.make_async_copy(x_hbm.at[src_v.at[i]], rows0, gsem0).wait()
            pltpu.sync_copy(rows0, acc.at[dst0], add=True)

            @pl.when(i + 2 < _NCHUNK)
            def _next_dst_even():
                pltpu.async_copy(dst3_hbm.at[wid, i + 2], dst0, dsem0)

        @pl.when(jnp.logical_not(even))
        def _drain_odd():
            pltpu.make_async_copy(dst3_hbm.at[wid, i], dst1, dsem1).wait()
            pltpu.make_async_copy(x_hbm.at[src_v.at[i]], rows1, gsem1).wait()
            pltpu.sync_copy(rows1, acc.at[dst1], add=True)

            @pl.when(i + 2 < _NCHUNK)
            def _next_dst_odd():
                pltpu.async_copy(dst3_hbm.at[wid, i + 2], dst1, dsem1)

        return carry

    lax.fori_loop(0, _NCHUNK, chunk, 0)
    plsc.subcore_barrier()
    _write_acc(acc, out_hbm, row0, c, s)


_CCH = 120
_CNFULL = _EW // _CCH
_CTAIL = _EW - _CNFULL * _CCH


def _cnt_body(ones_hbm, dst2_hbm, zeros_hbm, out_hbm, dst_v, rows, acc):
    c = lax.axis_index("c")
    s = lax.axis_index("s")
    wid = s * _NC + c
    row0 = pl.multiple_of(s * _RT, 8)
    _zero_acc(zeros_hbm, acc, row0, s)
    pltpu.sync_copy(ones_hbm, rows)
    pltpu.sync_copy(dst2_hbm.at[wid], dst_v)
    plsc.subcore_barrier()

    def chunk(i, carry):
        idx = dst_v.at[pl.ds(pl.multiple_of(i * _CCH, 8), _CCH)]
        pltpu.sync_copy(rows, acc.at[idx], add=True)
        return carry

    lax.fori_loop(0, _CNFULL, chunk, 0)
    tidx = dst_v.at[pl.ds(_CNFULL * _CCH, _CTAIL)]
    pltpu.sync_copy(rows.at[pl.ds(0, _CTAIL)], acc.at[tidx], add=True)
    plsc.subcore_barrier()
    _write_acc(acc, out_hbm, row0, c, s)


_cnt_call = pl.kernel(
    _cnt_body,
    out_type=jax.ShapeDtypeStruct((_NC, N, D), jnp.float32),
    mesh=_SC_MESH,
    scratch_types=[
        pltpu.VMEM((_EW,), jnp.int32),
        pltpu.VMEM((_CCH, D), jnp.float32),
        pltpu.VMEM_SHARED((N, D), jnp.float32),
    ],
)


_agg_call = pl.kernel(
    _agg_body,
    out_type=jax.ShapeDtypeStruct((_NC, N, D), jnp.float32),
    mesh=_SC_MESH,
    scratch_types=[
        pltpu.VMEM((_NCHUNK, _CH), jnp.int32),
        pltpu.VMEM((_CH,), jnp.int32),
        pltpu.VMEM((_CH,), jnp.int32),
        pltpu.VMEM((_CH, D), jnp.float32),
        pltpu.VMEM((_CH, D), jnp.float32),
        pltpu.VMEM_SHARED((N, D), jnp.float32),
        pltpu.SemaphoreType.DMA,
        pltpu.SemaphoreType.DMA,
        pltpu.SemaphoreType.DMA,
        pltpu.SemaphoreType.DMA,
    ],
)


def _pre_body(mix_ref, gs_ref, struct_ref, svd_ref, wst_ref, bs_ref,
              wgt_ref, bg_ref, lsg_ref, lsb_ref, lvg_ref, lvb_ref,
              sin_ref, tin_ref, out_ref):
    alpha = 1.0 / (1.0 + jnp.exp(-mix_ref[0, 0]))
    ctx = jnp.dot(gs_ref[...], wgt_ref[...],
                  preferred_element_type=jnp.float32) + bg_ref[...]
    sp = jnp.dot(struct_ref[...], wst_ref[...],
                 preferred_element_type=jnp.float32) + bs_ref[...]
    m = jnp.mean(sp, axis=-1, keepdims=True)
    v = jnp.mean((sp - m) ** 2, axis=-1, keepdims=True)
    sn = (sp - m) / jnp.sqrt(v + EPS) * lsg_ref[...] + lsb_ref[...]
    sv = svd_ref[...]
    m2 = jnp.mean(sv, axis=-1, keepdims=True)
    v2 = jnp.mean((sv - m2) ** 2, axis=-1, keepdims=True)
    vn = (sv - m2) / jnp.sqrt(v2 + EPS) * lvg_ref[...] + lvb_ref[...]
    x = alpha * sn + (1.0 - alpha) * vn + ctx
    x = x * sin_ref[...] + tin_ref[...]
    out_ref[...] = jnp.maximum(x, 0.0)


def _layer_body(p0_ref, p1_ref, inv_ref, x_ref, wlt_ref, wrt_ref, c_ref,
                out_ref, *, relu):
    agg = (p0_ref[...] + p1_ref[...]) * inv_ref[...]
    y = (jnp.dot(agg, wlt_ref[...], preferred_element_type=jnp.float32)
         + jnp.dot(x_ref[...], wrt_ref[...],
                   preferred_element_type=jnp.float32)
         + c_ref[...])
    if relu:
        y = jnp.maximum(y, 0.0)
    out_ref[...] = y


def _vec_spec():
    return pl.BlockSpec((1, D), lambda i: (0, 0))


def _pre_call(mix, gs, struct, svd, wst, bs, wgt, bg, lsg, lsb, lvg, lvb,
              sin, tin):
    return pl.pallas_call(
        _pre_body,
        grid=(GRID,),
        in_specs=[
            pl.BlockSpec((1, 1), lambda i: (0, 0)),
            pl.BlockSpec((1, 3), lambda i: (0, 0)),
            pl.BlockSpec((BLK, 6), lambda i: (i, 0)),
            pl.BlockSpec((BLK, D), lambda i: (i, 0)),
            pl.BlockSpec((6, D), lambda i: (0, 0)),
            _vec_spec(), pl.BlockSpec((3, D), lambda i: (0, 0)),
            _vec_spec(), _vec_spec(), _vec_spec(), _vec_spec(), _vec_spec(),
            _vec_spec(), _vec_spec(),
        ],
        out_specs=pl.BlockSpec((BLK, D), lambda i: (i, 0)),
        out_shape=jax.ShapeDtypeStruct((N, D), jnp.float32),
    )(mix, gs, struct, svd, wst, bs, wgt, bg, lsg, lsb, lvg, lvb, sin, tin)


def _layer_call(p0, p1, inv, x, wlt, wrt, c, relu):
    return pl.pallas_call(
        functools.partial(_layer_body, relu=relu),
        grid=(GRID,),
        in_specs=[
            pl.BlockSpec((BLK, D), lambda i: (i, 0)),
            pl.BlockSpec((BLK, D), lambda i: (i, 0)),
            pl.BlockSpec((BLK, 1), lambda i: (i, 0)),
            pl.BlockSpec((BLK, D), lambda i: (i, 0)),
            pl.BlockSpec((D, D), lambda i: (0, 0)),
            pl.BlockSpec((D, D), lambda i: (0, 0)),
            _vec_spec(),
        ],
        out_specs=pl.BlockSpec((BLK, D), lambda i: (i, 0)),
        out_shape=jax.ShapeDtypeStruct((N, D), jnp.float32),
    )(p0, p1, inv, x, wlt, wrt, c)


def _bn_fold(g, b, rm, rv):
    s = g / jnp.sqrt(rv + EPS)
    return s, b - rm * s


def kernel(struct, svd, graph_summary, Ws, bs, Wg, bg, ln_s_g, ln_s_b,
           ln_v_g, ln_v_b, mix, bn_in_g, bn_in_b, bn_in_rm, bn_in_rv,
           Wl0, bl0, Wr0, bn0_g, bn0_b, bn0_rm, bn0_rv,
           Wl1, bl1, Wr1, bn1_g, bn1_b, bn1_rm, bn1_rv,
           Wl2, bl2, Wr2, bn2_g, bn2_b, bn2_rm, bn2_rv, edge_index):
    f32 = jnp.float32
    sin, tin = _bn_fold(bn_in_g, bn_in_b, bn_in_rm, bn_in_rv)
    x = _pre_call(mix.reshape(1, 1), graph_summary.reshape(1, 3),
                  struct, svd, Ws.T, bs.reshape(1, D), Wg.T,
                  bg.reshape(1, D), ln_s_g.reshape(1, D),
                  ln_s_b.reshape(1, D), ln_v_g.reshape(1, D),
                  ln_v_b.reshape(1, D), sin.reshape(1, D),
                  tin.reshape(1, D))

    src3 = edge_index[0].reshape(_NW, _NCHUNK, _CH)
    dst3 = edge_index[1].reshape(_NW, _NCHUNK, _CH)
    dst2 = edge_index[1].reshape(_NW, _EW)
    zeros = jnp.zeros((N, D), f32)

    layers = [(Wl0, bl0, Wr0, bn0_g, bn0_b, bn0_rm, bn0_rv),
              (Wl1, bl1, Wr1, bn1_g, bn1_b, bn1_rm, bn1_rv),
              (Wl2, bl2, Wr2, bn2_g, bn2_b, bn2_rm, bn2_rv)]
    inv = None
    for i, (Wl, bl, Wr, g, b, rm, rv) in enumerate(layers):
        s, t = _bn_fold(g, b, rm, rv)
        wlt = Wl.T * s[None, :]
        wrt = Wr.T * s[None, :]
        c = (bl * s + t).reshape(1, D)
        if i == 0:
            cparts = _cnt_call(jnp.ones((_CCH, D), f32), dst2, zeros)
            cnt = cparts[0, :, 0] + cparts[1, :, 0]
            inv = (1.0 / jnp.maximum(cnt, 1.0)).reshape(N, 1)
        parts = _agg_call(x, src3, dst3, zeros)
        x = _layer_call(parts[0], parts[1], inv, x, wlt, wrt, c,
                        relu=(i < 2))
    return x


# speedup vs baseline: 9.2067x; 1.0221x over previous
import functools

import jax
import jax.numpy as jnp
from jax import lax
from jax.experimental import pallas as pl
from jax.experimental.pallas import tpu as pltpu
from jax.experimental.pallas import tpu_sc as plsc

N = 10000
E = 320000
D = 128
EPS = 1e-5
BLK = 1000
GRID = N // BLK

_NC = 2
_NS = 16
_NW = _NC * _NS
_EW = E // _NW
_CH = 80
_NCHUNK = _EW // _CH
_RT = 624
_RTAIL = N - _RT * _NS

_SC_MESH = plsc.VectorSubcoreMesh(core_axis_name="c", subcore_axis_name="s",
                                  num_cores=_NC, num_subcores=_NS)


def _zero_acc(zeros_hbm, acc, row0, s):
    pltpu.sync_copy(zeros_hbm.at[pl.ds(row0, _RT)], acc.at[pl.ds(row0, _RT)])

    @pl.when(s == _NS - 1)
    def _zero_tail():
        pltpu.sync_copy(zeros_hbm.at[pl.ds(_RT * _NS, _RTAIL)],
                        acc.at[pl.ds(_RT * _NS, _RTAIL)])


def _write_acc(acc, out_hbm, row0, c, s):
    pltpu.sync_copy(acc.at[pl.ds(row0, _RT)],
                    out_hbm.at[c, pl.ds(row0, _RT)])

    @pl.when(s == _NS - 1)
    def _write_tail():
        pltpu.sync_copy(acc.at[pl.ds(_RT * _NS, _RTAIL)],
                        out_hbm.at[c, pl.ds(_RT * _NS, _RTAIL)])


def _agg_body(x_hbm, src3_hbm, dst3_hbm, zeros_hbm, out_hbm, src_v,
              dst0, dst1, rows0, rows1, acc, gsem0, gsem0b, gsem1, gsem1b,
              dsem0, dsem1):
    c = lax.axis_index("c")
    s = lax.axis_index("s")
    wid = s * _NC + c
    row0 = pl.multiple_of(s * _RT, 8)
    _zero_acc(zeros_hbm, acc, row0, s)
    pltpu.sync_copy(src3_hbm.at[wid], src_v)
    plsc.subcore_barrier()

    _H = _CH // 2

    def _gather(nxt, rows, sa, sb):
        pltpu.async_copy(x_hbm.at[src_v.at[nxt, pl.ds(0, _H)]],
                         rows.at[pl.ds(0, _H)], sa)
        pltpu.async_copy(x_hbm.at[src_v.at[nxt, pl.ds(_H, _H)]],
                         rows.at[pl.ds(_H, _H)], sb)

    def _gwait(i, rows, sa, sb):
        pltpu.make_async_copy(x_hbm.at[src_v.at[i, pl.ds(0, _H)]],
                              rows.at[pl.ds(0, _H)], sa).wait()
        pltpu.make_async_copy(x_hbm.at[src_v.at[i, pl.ds(_H, _H)]],
                              rows.at[pl.ds(_H, _H)], sb).wait()

    _gather(0, rows0, gsem0, gsem0b)
    pltpu.async_copy(dst3_hbm.at[wid, 0], dst0, dsem0)
    pltpu.async_copy(dst3_hbm.at[wid, 1], dst1, dsem1)

    def chunk(i, carry):
        even = i % 2 == 0
        nxt = i + 1

        @pl.when(jnp.logical_and(nxt < _NCHUNK, even))
        def _start_odd():
            _gather(nxt, rows1, gsem1, gsem1b)

        @pl.when(jnp.logical_and(nxt < _NCHUNK, jnp.logical_not(even)))
        def _start_even():
            _gather(nxt, rows0, gsem0, gsem0b)

        @pl.when(even)
        def _drain_even():
            pltpu.make_async_copy(dst3_hbm.at[wid, i], dst0, dsem0).wait()
            _gwait(i, rows0, gsem0, gsem0b)
            pltpu.sync_copy(rows0, acc.at[dst0], add=True)

            @pl.when(i + 2 < _NCHUNK)
            def _next_dst_even():
                pltpu.async_copy(dst3_hbm.at[wid, i + 2], dst0, dsem0)

        @pl.when(jnp.logical_not(even))
        def _drain_odd():
            pltpu.make_async_copy(dst3_hbm.at[wid, i], dst1, dsem1).wait()
            _gwait(i, rows1, gsem1, gsem1b)
            pltpu.sync_copy(rows1, acc.at[dst1], add=True)

            @pl.when(i + 2 < _NCHUNK)
            def _next_dst_odd():
                pltpu.async_copy(dst3_hbm.at[wid, i + 2], dst1, dsem1)

        return carry

    lax.fori_loop(0, _NCHUNK, chunk, 0)
    plsc.subcore_barrier()
    _write_acc(acc, out_hbm, row0, c, s)


_CCH = 120
_CNFULL = _EW // _CCH
_CTAIL = _EW - _CNFULL * _CCH


def _cnt_body(ones_hbm, dst2_hbm, zeros_hbm, out_hbm, dst_v, rows, acc):
    c = lax.axis_index("c")
    s = lax.axis_index("s")
    wid = s * _NC + c
    row0 = pl.multiple_of(s * _RT, 8)
    _zero_acc(zeros_hbm, acc, row0, s)
    pltpu.sync_copy(ones_hbm, rows)
    pltpu.sync_copy(dst2_hbm.at[wid], dst_v)
    plsc.subcore_barrier()

    def chunk(i, carry):
        idx = dst_v.at[pl.ds(pl.multiple_of(i * _CCH, 8), _CCH)]
        pltpu.sync_copy(rows, acc.at[idx], add=True)
        return carry

    lax.fori_loop(0, _CNFULL, chunk, 0)
    tidx = dst_v.at[pl.ds(_CNFULL * _CCH, _CTAIL)]
    pltpu.sync_copy(rows.at[pl.ds(0, _CTAIL)], acc.at[tidx], add=True)
    plsc.subcore_barrier()
    _write_acc(acc, out_hbm, row0, c, s)


_cnt_call = pl.kernel(
    _cnt_body,
    out_type=jax.ShapeDtypeStruct((_NC, N, D), jnp.float32),
    mesh=_SC_MESH,
    scratch_types=[
        pltpu.VMEM((_EW,), jnp.int32),
        pltpu.VMEM((_CCH, D), jnp.float32),
        pltpu.VMEM_SHARED((N, D), jnp.float32),
    ],
)


_agg_call = pl.kernel(
    _agg_body,
    out_type=jax.ShapeDtypeStruct((_NC, N, D), jnp.float32),
    mesh=_SC_MESH,
    scratch_types=[
        pltpu.VMEM((_NCHUNK, _CH), jnp.int32),
        pltpu.VMEM((_CH,), jnp.int32),
        pltpu.VMEM((_CH,), jnp.int32),
        pltpu.VMEM((_CH, D), jnp.float32),
        pltpu.VMEM((_CH, D), jnp.float32),
        pltpu.VMEM_SHARED((N, D), jnp.float32),
        pltpu.SemaphoreType.DMA,
        pltpu.SemaphoreType.DMA,
        pltpu.SemaphoreType.DMA,
        pltpu.SemaphoreType.DMA,
        pltpu.SemaphoreType.DMA,
        pltpu.SemaphoreType.DMA,
    ],
)


def _pre_body(mix_ref, gs_ref, struct_ref, svd_ref, wst_ref, bs_ref,
              wgt_ref, bg_ref, lsg_ref, lsb_ref, lvg_ref, lvb_ref,
              sin_ref, tin_ref, out_ref):
    alpha = 1.0 / (1.0 + jnp.exp(-mix_ref[0, 0]))
    ctx = jnp.dot(gs_ref[...], wgt_ref[...],
                  preferred_element_type=jnp.float32) + bg_ref[...]
    sp = jnp.dot(struct_ref[...], wst_ref[...],
                 preferred_element_type=jnp.float32) + bs_ref[...]
    m = jnp.mean(sp, axis=-1, keepdims=True)
    v = jnp.mean((sp - m) ** 2, axis=-1, keepdims=True)
    sn = (sp - m) / jnp.sqrt(v + EPS) * lsg_ref[...] + lsb_ref[...]
    sv = svd_ref[...]
    m2 = jnp.mean(sv, axis=-1, keepdims=True)
    v2 = jnp.mean((sv - m2) ** 2, axis=-1, keepdims=True)
    vn = (sv - m2) / jnp.sqrt(v2 + EPS) * lvg_ref[...] + lvb_ref[...]
    x = alpha * sn + (1.0 - alpha) * vn + ctx
    x = x * sin_ref[...] + tin_ref[...]
    out_ref[...] = jnp.maximum(x, 0.0)


def _layer_body(p0_ref, p1_ref, inv_ref, x_ref, wlt_ref, wrt_ref, c_ref,
                out_ref, *, relu):
    agg = (p0_ref[...] + p1_ref[...]) * inv_ref[...]
    y = (jnp.dot(agg, wlt_ref[...], preferred_element_type=jnp.float32)
         + jnp.dot(x_ref[...], wrt_ref[...],
                   preferred_element_type=jnp.float32)
         + c_ref[...])
    if relu:
        y = jnp.maximum(y, 0.0)
    out_ref[...] = y


def _vec_spec():
    return pl.BlockSpec((1, D), lambda i: (0, 0))


def _pre_call(mix, gs, struct, svd, wst, bs, wgt, bg, lsg, lsb, lvg, lvb,
              sin, tin):
    return pl.pallas_call(
        _pre_body,
        grid=(GRID,),
        in_specs=[
            pl.BlockSpec((1, 1), lambda i: (0, 0)),
            pl.BlockSpec((1, 3), lambda i: (0, 0)),
            pl.BlockSpec((BLK, 6), lambda i: (i, 0)),
            pl.BlockSpec((BLK, D), lambda i: (i, 0)),
            pl.BlockSpec((6, D), lambda i: (0, 0)),
            _vec_spec(), pl.BlockSpec((3, D), lambda i: (0, 0)),
            _vec_spec(), _vec_spec(), _vec_spec(), _vec_spec(), _vec_spec(),
            _vec_spec(), _vec_spec(),
        ],
        out_specs=pl.BlockSpec((BLK, D), lambda i: (i, 0)),
        out_shape=jax.ShapeDtypeStruct((N, D), jnp.float32),
    )(mix, gs, struct, svd, wst, bs, wgt, bg, lsg, lsb, lvg, lvb, sin, tin)


def _layer_call(p0, p1, inv, x, wlt, wrt, c, relu):
    return pl.pallas_call(
        functools.partial(_layer_body, relu=relu),
        grid=(GRID,),
        in_specs=[
            pl.BlockSpec((BLK, D), lambda i: (i, 0)),
            pl.BlockSpec((BLK, D), lambda i: (i, 0)),
            pl.BlockSpec((BLK, 1), lambda i: (i, 0)),
            pl.BlockSpec((BLK, D), lambda i: (i, 0)),
            pl.BlockSpec((D, D), lambda i: (0, 0)),
            pl.BlockSpec((D, D), lambda i: (0, 0)),
            _vec_spec(),
        ],
        out_specs=pl.BlockSpec((BLK, D), lambda i: (i, 0)),
        out_shape=jax.ShapeDtypeStruct((N, D), jnp.float32),
    )(p0, p1, inv, x, wlt, wrt, c)


def _bn_fold(g, b, rm, rv):
    s = g / jnp.sqrt(rv + EPS)
    return s, b - rm * s


def kernel(struct, svd, graph_summary, Ws, bs, Wg, bg, ln_s_g, ln_s_b,
           ln_v_g, ln_v_b, mix, bn_in_g, bn_in_b, bn_in_rm, bn_in_rv,
           Wl0, bl0, Wr0, bn0_g, bn0_b, bn0_rm, bn0_rv,
           Wl1, bl1, Wr1, bn1_g, bn1_b, bn1_rm, bn1_rv,
           Wl2, bl2, Wr2, bn2_g, bn2_b, bn2_rm, bn2_rv, edge_index):
    f32 = jnp.float32
    sin, tin = _bn_fold(bn_in_g, bn_in_b, bn_in_rm, bn_in_rv)
    x = _pre_call(mix.reshape(1, 1), graph_summary.reshape(1, 3),
                  struct, svd, Ws.T, bs.reshape(1, D), Wg.T,
                  bg.reshape(1, D), ln_s_g.reshape(1, D),
                  ln_s_b.reshape(1, D), ln_v_g.reshape(1, D),
                  ln_v_b.reshape(1, D), sin.reshape(1, D),
                  tin.reshape(1, D))

    src3 = edge_index[0].reshape(_NW, _NCHUNK, _CH)
    dst3 = edge_index[1].reshape(_NW, _NCHUNK, _CH)
    dst2 = edge_index[1].reshape(_NW, _EW)
    zeros = jnp.zeros((N, D), f32)

    layers = [(Wl0, bl0, Wr0, bn0_g, bn0_b, bn0_rm, bn0_rv),
              (Wl1, bl1, Wr1, bn1_g, bn1_b, bn1_rm, bn1_rv),
              (Wl2, bl2, Wr2, bn2_g, bn2_b, bn2_rm, bn2_rv)]
    inv = None
    for i, (Wl, bl, Wr, g, b, rm, rv) in enumerate(layers):
        s, t = _bn_fold(g, b, rm, rv)
        wlt = Wl.T * s[None, :]
        wrt = Wr.T * s[None, :]
        c = (bl * s + t).reshape(1, D)
        if i == 0:
            cparts = _cnt_call(jnp.ones((_CCH, D), f32), dst2, zeros)
            cnt = cparts[0, :, 0] + cparts[1, :, 0]
            inv = (1.0 / jnp.maximum(cnt, 1.0)).reshape(N, 1)
        parts = _agg_call(x, src3, dst3, zeros)
        x = _layer_call(parts[0], parts[1], inv, x, wlt, wrt, c,
                        relu=(i < 2))
    return x
